# baseline (speedup 1.0000x reference)
import functools

import jax
import jax.numpy as jnp
import numpy as np
from jax import lax
from jax.experimental import pallas as pl
from jax.experimental.pallas import tpu as pltpu
from jax.experimental.pallas import tpu_sc as plsc

VOCAB = 1000
D = 128
B = 1024
L = 200

NC = 2
NS = 16
NW = NC * NS
ROWS_PER_W = B // NW
HALF = L // 2
NBUF = 3


def _positional_encoding(length, channels):
    ch = int(np.ceil(channels / 2) * 2)
    inv_freq = 1.0 / (10000.0 ** (np.arange(0, ch, 2, dtype=np.float64) / ch))
    pos = np.arange(length, dtype=np.float64)
    sin_inp = pos[:, None] * inv_freq[None, :]
    emb = np.concatenate([np.sin(sin_inp), np.cos(sin_inp)], axis=-1)
    return jnp.asarray(emb[:, :channels], dtype=jnp.float32)


_mesh = plsc.VectorSubcoreMesh(core_axis_name="c", subcore_axis_name="s")


@functools.partial(
    pl.kernel,
    mesh=_mesh,
    out_type=jax.ShapeDtypeStruct((B, L, D), jnp.float32),
    scratch_types=[
        pltpu.VMEM((2 * ROWS_PER_W, HALF), jnp.int32),
        pltpu.VMEM((L, D), jnp.float32),
        pltpu.VMEM((NBUF, L, D), jnp.float32),
        pltpu.VMEM_SHARED((VOCAB, D), jnp.float32),
    ]
    + [pltpu.SemaphoreType.DMA] * (2 * NBUF),
)
def _emb_kernel(
    token_hbm, table_hbm, pe_hbm, out_hbm, idx_v, pe_v, rows_v, table_sh, *sems
):
    gsems = sems[:NBUF]
    osems = sems[NBUF:]
    cid = lax.axis_index("c")
    sid = lax.axis_index("s")
    wid = sid * NC + cid
    row_base = wid * ROWS_PER_W

    @pl.when(sid == 0)
    def _():
        pltpu.sync_copy(table_hbm, table_sh)

    pltpu.sync_copy(token_hbm.at[wid], idx_v)
    pltpu.sync_copy(pe_hbm, pe_v)
    plsc.subcore_barrier()

    def fire_g(c, b):
        pltpu.async_copy(
            table_sh.at[idx_v.at[2 * c]], rows_v.at[b, pl.ds(0, HALF)], gsems[b]
        )
        pltpu.async_copy(
            table_sh.at[idx_v.at[2 * c + 1]],
            rows_v.at[b, pl.ds(HALF, HALF)],
            gsems[b],
        )

    def wait_g(b):
        for h in range(2):
            pltpu.make_async_copy(
                table_sh.at[idx_v.at[0]],
                rows_v.at[b, pl.ds(h * HALF, HALF)],
                gsems[b],
            ).wait()

    def fire_o(c, b):
        pltpu.async_copy(rows_v.at[b], out_hbm.at[row_base + c], osems[b])

    def wait_o(b):
        pltpu.make_async_copy(rows_v.at[b], out_hbm.at[0], osems[b]).wait()

    def add_pe(b):
        def add_step(rr, carry):
            for k in range(D // 16):
                plsc.addupdate(
                    rows_v.at[b, rr, pl.ds(k * 16, 16)],
                    pe_v[rr, pl.ds(k * 16, 16)],
                )
            return carry

        lax.fori_loop(0, L, add_step, 0, unroll=4)

    def chunk_body(c, b, wait_prev_out, fire_next):
        wait_g(b)
        if fire_next:
            if wait_prev_out:
                wait_o((b + 2) % NBUF)
            fire_g(c + 2, (b + 2) % NBUF)
        add_pe(b)
        fire_o(c, b)

    fire_g(0, 0)
    fire_g(1, 1)

    chunk_body(0, 0, wait_prev_out=False, fire_next=True)
    chunk_body(1, 1, wait_prev_out=True, fire_next=True)
    chunk_body(2, 2, wait_prev_out=True, fire_next=True)

    def group_body(g, carry):
        for b in range(NBUF):
            chunk_body(g * NBUF + b, b, wait_prev_out=True, fire_next=True)
        return carry

    lax.fori_loop(1, ROWS_PER_W // NBUF, group_body, 0)

    chunk_body(ROWS_PER_W - 2, 0, wait_prev_out=False, fire_next=False)
    chunk_body(ROWS_PER_W - 1, 1, wait_prev_out=False, fire_next=False)

    for b in range(NBUF):
        wait_o(b)


@jax.jit
def kernel(bar, pos, token, dur, phrase, token_emb_weight):
    del bar, pos, dur, phrase
    pe = _positional_encoding(L, D)
    tok = token.reshape(NW, 2 * ROWS_PER_W, HALF)
    return _emb_kernel(tok, token_emb_weight, pe)


# speedup vs baseline: 1.2446x; 1.2446x over previous
import functools

import jax
import jax.numpy as jnp
import numpy as np
from jax import lax
from jax.experimental import pallas as pl
from jax.experimental.pallas import tpu as pltpu
from jax.experimental.pallas import tpu_sc as plsc

VOCAB = 1000
D = 128
B = 1024
L = 200

NC = 2
NS = 16
NW = NC * NS
ROWS_PER_W = B // NW
HALF = L // 2
NBUF = 3


def _positional_encoding(length, channels):
    ch = int(np.ceil(channels / 2) * 2)
    inv_freq = 1.0 / (10000.0 ** (np.arange(0, ch, 2, dtype=np.float64) / ch))
    pos = np.arange(length, dtype=np.float64)
    sin_inp = pos[:, None] * inv_freq[None, :]
    emb = np.concatenate([np.sin(sin_inp), np.cos(sin_inp)], axis=-1)
    return jnp.asarray(emb[:, :channels], dtype=jnp.float32)


_mesh = plsc.VectorSubcoreMesh(core_axis_name="c", subcore_axis_name="s")


@functools.partial(
    pl.kernel,
    mesh=_mesh,
    out_type=jax.ShapeDtypeStruct((B, L, D), jnp.float32),
    scratch_types=[
        pltpu.VMEM((2 * ROWS_PER_W, HALF), jnp.int32),
        pltpu.VMEM((L, D), jnp.float32),
        pltpu.VMEM((NBUF, L, D), jnp.float32),
        pltpu.VMEM_SHARED((VOCAB, D), jnp.float32),
    ]
    + [pltpu.SemaphoreType.DMA] * (2 * NBUF),
)
def _emb_kernel(
    token_hbm, table_hbm, pe_hbm, out_hbm, idx_v, pe_v, rows_v, table_sh, *sems
):
    gsems = sems[:NBUF]
    osems = sems[NBUF:]
    cid = lax.axis_index("c")
    sid = lax.axis_index("s")
    wid = sid * NC + cid
    row_base = wid * ROWS_PER_W

    @pl.when(sid == 0)
    def _():
        pltpu.sync_copy(table_hbm, table_sh)

    pltpu.sync_copy(token_hbm.at[wid], idx_v)
    pltpu.sync_copy(pe_hbm, pe_v)
    plsc.subcore_barrier()

    def fire_g(c, b):
        pltpu.async_copy(
            table_sh.at[idx_v.at[2 * c]], rows_v.at[b, pl.ds(0, HALF)], gsems[b]
        )
        pltpu.async_copy(
            table_sh.at[idx_v.at[2 * c + 1]],
            rows_v.at[b, pl.ds(HALF, HALF)],
            gsems[b],
        )

    def wait_g(b):
        for h in range(2):
            pltpu.make_async_copy(
                table_sh.at[idx_v.at[0]],
                rows_v.at[b, pl.ds(h * HALF, HALF)],
                gsems[b],
            ).wait()

    def fire_o(c, b):
        pltpu.async_copy(rows_v.at[b], out_hbm.at[row_base + c], osems[b])

    def wait_o(b):
        pltpu.make_async_copy(rows_v.at[b], out_hbm.at[0], osems[b]).wait()

    def add_pe(b):
        def add_step(rr, carry):
            for k in range(D // 16):
                plsc.addupdate(
                    rows_v.at[b, rr, pl.ds(k * 16, 16)],
                    pe_v[rr, pl.ds(k * 16, 16)],
                )
            return carry

        lax.fori_loop(0, L, add_step, 0, unroll=4)

    def chunk_body(c, b, wait_prev_out, fire_next):
        wait_g(b)
        if wait_prev_out:
            wait_o((b + 1) % NBUF)
        if fire_next:
            fire_g(c + 1, (b + 1) % NBUF)
        add_pe(b)
        fire_o(c, b)

    fire_g(0, 0)

    chunk_body(0, 0, wait_prev_out=False, fire_next=True)
    chunk_body(1, 1, wait_prev_out=False, fire_next=True)
    chunk_body(2, 2, wait_prev_out=True, fire_next=True)

    def group_body(g, carry):
        for b in range(NBUF):
            chunk_body(g * NBUF + b, b, wait_prev_out=True, fire_next=True)
        return carry

    lax.fori_loop(1, ROWS_PER_W // NBUF, group_body, 0)

    chunk_body(ROWS_PER_W - 2, 0, wait_prev_out=True, fire_next=True)
    chunk_body(ROWS_PER_W - 1, 1, wait_prev_out=False, fire_next=False)

    for b in range(NBUF):
        wait_o(b)


@jax.jit
def kernel(bar, pos, token, dur, phrase, token_emb_weight):
    del bar, pos, dur, phrase
    pe = _positional_encoding(L, D)
    tok = token.reshape(NW, 2 * ROWS_PER_W, HALF)
    return _emb_kernel(tok, token_emb_weight, pe)
